# Initial kernel scaffold; baseline (speedup 1.0000x reference)
import jax
import jax.numpy as jnp
from jax.experimental import pallas as pl


def kernel(user_ids, item_ids, user_cf_weight, item_cf_weight, raw_user_profiles, article_content_embeddings, proj_W, proj_b, ln_gamma, ln_beta):
    raise NotImplementedError("write your pallas kernel here")


# speedup vs baseline: 1.9504x; 1.9504x over previous
import functools

import jax
import jax.numpy as jnp
from jax import lax
from jax.experimental import pallas as pl
from jax.experimental.pallas import tpu as pltpu
from jax.experimental.pallas import tpu_sc as plsc

BATCH = 16384
CF_DIM = 64
CD = 256
ALPHA = 0.5

NC = 2
NS = 16
NW = NC * NS
BPW = BATCH // NW
SUB = 128
NSUB = BPW // SUB
LANES = 16


@functools.cache
def _make_sc_gather():
    mesh = plsc.VectorSubcoreMesh(core_axis_name="c", subcore_axis_name="s",
                                  num_cores=NC, num_subcores=NS)

    @functools.partial(
        pl.kernel,
        out_type=[
            jax.ShapeDtypeStruct((BATCH, 2 * CF_DIM), jnp.float32),
            jax.ShapeDtypeStruct((BATCH, 2 * CF_DIM), jnp.float32),
            jax.ShapeDtypeStruct((BATCH, CD), jnp.float32),
            jax.ShapeDtypeStruct((BATCH, CD), jnp.float32),
        ],
        mesh=mesh,
        scratch_types=[
            pltpu.VMEM((BPW,), jnp.int32),
            pltpu.VMEM((BPW,), jnp.int32),
            pltpu.VMEM((BPW,), jnp.int32),
            pltpu.VMEM((BPW,), jnp.int32),
            pltpu.VMEM((SUB, 2 * CF_DIM), jnp.float32),
            pltpu.VMEM((SUB, 2 * CF_DIM), jnp.float32),
            pltpu.VMEM((SUB, CD), jnp.float32),
            pltpu.VMEM((SUB, CD), jnp.float32),
            pltpu.SemaphoreType.DMA,
        ],
    )
    def _sc_gather(uids, iids, ucf2, icf2, uprof, icont,
                   ucf_out, icf_out, uprof_out, icont_out,
                   uid_v, iid_v, uhalf_v, ihalf_v,
                   ubuf128, ibuf128, ubuf256, ibuf256, sem):
        wid = lax.axis_index("s") * NC + lax.axis_index("c")
        base = wid * BPW
        pltpu.sync_copy(uids.at[pl.ds(base, BPW)], uid_v)
        pltpu.sync_copy(iids.at[pl.ds(base, BPW)], iid_v)

        def shift_body(i, _):
            uhalf_v[pl.ds(i * LANES, LANES)] = (
                uid_v[pl.ds(i * LANES, LANES)] >> 1)
            ihalf_v[pl.ds(i * LANES, LANES)] = (
                iid_v[pl.ds(i * LANES, LANES)] >> 1)
            return 0

        lax.fori_loop(0, BPW // LANES, shift_body, 0)

        for c in range(NSUB):
            o = c * SUB
            pltpu.async_copy(ucf2.at[uhalf_v.at[pl.ds(o, SUB)]], ubuf128, sem).wait()
            pltpu.async_copy(icf2.at[ihalf_v.at[pl.ds(o, SUB)]], ibuf128, sem).wait()
            pltpu.async_copy(uprof.at[uid_v.at[pl.ds(o, SUB)]], ubuf256, sem).wait()
            pltpu.async_copy(icont.at[iid_v.at[pl.ds(o, SUB)]], ibuf256, sem).wait()
            pltpu.sync_copy(ubuf128, ucf_out.at[pl.ds(base + o, SUB)])
            pltpu.sync_copy(ibuf128, icf_out.at[pl.ds(base + o, SUB)])
            pltpu.sync_copy(ubuf256, uprof_out.at[pl.ds(base + o, SUB)])
            pltpu.sync_copy(ibuf256, icont_out.at[pl.ds(base + o, SUB)])

    return _sc_gather


BLK = 1024


def _tc_body(uids_ref, iids_ref, ucf_ref, icf_ref, uprof_ref, icont_ref,
             w_ref, b_ref, g_ref, beta_ref, out_ref):
    u = uprof_ref[...]
    h = jnp.dot(u, w_ref[...], preferred_element_type=jnp.float32)
    h = h + b_ref[...]
    mu = jnp.mean(h, axis=1, keepdims=True)
    var = jnp.mean((h - mu) * (h - mu), axis=1, keepdims=True)
    hn = (h - mu) * lax.rsqrt(var + 1e-5) * g_ref[...] + beta_ref[...]
    hg = hn * 0.5 * (1.0 + lax.erf(hn * 0.7071067811865476))
    content = jnp.sum(hg * icont_ref[...], axis=1)
    u_odd = (uids_ref[...] & 1)[:, None] == 1
    i_odd = (iids_ref[...] & 1)[:, None] == 1
    ucf = jnp.where(u_odd, ucf_ref[:, CF_DIM:], ucf_ref[:, :CF_DIM])
    icf = jnp.where(i_odd, icf_ref[:, CF_DIM:], icf_ref[:, :CF_DIM])
    cf = jnp.sum(ucf * icf, axis=1)
    out_ref[...] = ALPHA * cf + (1.0 - ALPHA) * content


_tc_score = pl.pallas_call(
    _tc_body,
    grid=(BATCH // BLK,),
    in_specs=[
        pl.BlockSpec((BLK,), lambda i: (i,)),
        pl.BlockSpec((BLK,), lambda i: (i,)),
        pl.BlockSpec((BLK, 2 * CF_DIM), lambda i: (i, 0)),
        pl.BlockSpec((BLK, 2 * CF_DIM), lambda i: (i, 0)),
        pl.BlockSpec((BLK, CD), lambda i: (i, 0)),
        pl.BlockSpec((BLK, CD), lambda i: (i, 0)),
        pl.BlockSpec((CD, CD), lambda i: (0, 0)),
        pl.BlockSpec((1, CD), lambda i: (0, 0)),
        pl.BlockSpec((1, CD), lambda i: (0, 0)),
        pl.BlockSpec((1, CD), lambda i: (0, 0)),
    ],
    out_specs=pl.BlockSpec((BLK,), lambda i: (i,)),
    out_shape=jax.ShapeDtypeStruct((BATCH,), jnp.float32),
)


def kernel(user_ids, item_ids, user_cf_weight, item_cf_weight,
           raw_user_profiles, article_content_embeddings,
           proj_W, proj_b, ln_gamma, ln_beta):
    ucf2 = user_cf_weight.reshape(-1, 2 * CF_DIM)
    icf2 = item_cf_weight.reshape(-1, 2 * CF_DIM)
    ucf_g, icf_g, uprof_g, icont_g = _make_sc_gather()(
        user_ids, item_ids, ucf2, icf2,
        raw_user_profiles, article_content_embeddings)
    return _tc_score(user_ids, item_ids, ucf_g, icf_g, uprof_g, icont_g,
                     proj_W, proj_b.reshape(1, CD), ln_gamma.reshape(1, CD),
                     ln_beta.reshape(1, CD))
